# baseline (speedup 1.0000x reference)
import functools

import jax
import jax.numpy as jnp
from jax import lax
from jax.experimental import pallas as pl
from jax.experimental.pallas import tpu as pltpu
from jax.experimental.pallas import tpu_sc as plsc

VOCAB = 1000000
HIDDEN = 64
B = 16384
L = 50

NUM_CORES = 2
NUM_SUBCORES = 16
NW = NUM_CORES * NUM_SUBCORES

CB = 400
NBLK = VOCAB // CB
VBLK = 80

BW = B // NW
TBW = BW // 128
PLANE = HIDDEN * B

_mesh = plsc.VectorSubcoreMesh(core_axis_name="c", subcore_axis_name="s")


def _wid():
    return lax.axis_index("s") * NUM_CORES + lax.axis_index("c")


@functools.partial(
    pl.kernel,
    mesh=_mesh,
    out_type=jax.ShapeDtypeStruct((VOCAB * HIDDEN,), jnp.float32),
    scratch_types=[
        pltpu.VMEM((HIDDEN, CB), jnp.float32),
        pltpu.VMEM((HIDDEN, CB), jnp.float32),
        pltpu.VMEM((CB * HIDDEN,), jnp.float32),
        pltpu.VMEM((CB * HIDDEN,), jnp.float32),
        pltpu.SemaphoreType.DMA,
        pltpu.SemaphoreType.DMA,
        pltpu.SemaphoreType.DMA,
        pltpu.SemaphoreType.DMA,
    ],
    compiler_params=pltpu.CompilerParams(use_tc_tiling_on_sc=False, needs_layout_passes=False),
)
def _relayout(tT_hbm, tlin_hbm, b0, b1, t0, t1, r0, r1, w0, w1):
    wid = _wid()
    buf = (b0, b1)
    bufT = (t0, t1)
    rsem = (r0, r1)
    wsem = (w0, w1)
    iota64 = lax.iota(jnp.int32, 16) * HIDDEN

    def c0(i):
        blk = jnp.minimum(wid + i * NW, NBLK - 1)
        return pl.multiple_of(blk * CB, 8)

    def start_read(i, s):
        pltpu.async_copy(tT_hbm.at[:, pl.ds(c0(i), CB)], buf[s], rsem[s])

    def wait_read(i, s):
        pltpu.make_async_copy(tT_hbm.at[:, pl.ds(c0(i), CB)], buf[s], rsem[s]).wait()

    def transpose(s):
        def body(g, carry):
            for cu in range(4):
                c = g * 4 + cu
                base = jnp.full((16,), c, jnp.int32)
                for rr in range(CB // 16):
                    vec = buf[s][c, pl.ds(rr * 16, 16)]
                    idxv = iota64 + (base + rr * 16 * HIDDEN)
                    plsc.store_scatter(bufT[s], [idxv], vec)
            return carry
        lax.fori_loop(0, HIDDEN // 4, body, 0)

    def start_write(i, s):
        pltpu.async_copy(bufT[s], tlin_hbm.at[pl.ds(c0(i) * HIDDEN, CB * HIDDEN)], wsem[s])

    def wait_write(i, s):
        pltpu.make_async_copy(bufT[s], tlin_hbm.at[pl.ds(c0(i) * HIDDEN, CB * HIDDEN)],
                              wsem[s]).wait()

    start_read(0, 0)
    wait_read(0, 0)
    start_read(1, 1)
    transpose(0)
    start_write(0, 0)
    wait_read(1, 1)
    start_read(2, 0)
    transpose(1)
    start_write(1, 1)

    def body(g, carry):
        for u in range(2):
            i = g * 2 + u
            wait_read(i, u)
            start_read(i + 1, 1 - u)
            wait_write(i - 2, u)
            transpose(u)
            start_write(i, u)
        return carry

    lax.fori_loop(1, VBLK // 2, body, 0)

    wait_write(VBLK - 2, 0)
    wait_write(VBLK - 1, 1)
    wait_read(VBLK, 0)


@functools.partial(
    pl.kernel,
    mesh=_mesh,
    out_type=jax.ShapeDtypeStruct((L, PLANE), jnp.float32),
    scratch_types=[
        pltpu.VMEM((BW,), jnp.int32),
        pltpu.VMEM((BW,), jnp.int32),
        pltpu.VMEM((BW, HIDDEN), jnp.float32),
        pltpu.VMEM((BW, HIDDEN), jnp.float32),
        pltpu.VMEM((BW * HIDDEN,), jnp.float32),
        pltpu.SemaphoreType.DMA,
        pltpu.SemaphoreType.DMA,
        pltpu.SemaphoreType.DMA,
        pltpu.SemaphoreType.DMA,
        pltpu.SemaphoreType.DMA,
    ],
    compiler_params=pltpu.CompilerParams(use_tc_tiling_on_sc=False, needs_layout_passes=False),
)
def _gather(xT_hbm, tlin_hbm, out_hbm,
            i0, i1, rowsA, rowsB, rT, is0, is1, gs0, gs1, ssem):
    wid = _wid()
    base = pl.multiple_of(wid * BW, 8)
    idx = (i0, i1)
    rows = (rowsA, rowsB)
    isem = (is0, is1)
    gsem = (gs0, gs1)
    cpat = []
    ii = lax.iota(jnp.int32, 16)
    for cc in range(4):
        cv = ii + cc * 16
        cpat.append((cv // 8) * (TBW * 1024) + (cv % 8) * 128)

    def start_idx(l, s):
        lc = jnp.minimum(l, L - 1)
        pltpu.async_copy(xT_hbm.at[lc, pl.ds(base, BW)], idx[s], isem[s])

    def wait_idx(s):
        pltpu.make_async_copy(xT_hbm.at[0, pl.ds(base, BW)], idx[s], isem[s]).wait()

    def start_gather(s):
        pltpu.async_copy(tlin_hbm.at[idx[s]], rows[s], gsem[s])

    def wait_gather(s):
        pltpu.make_async_copy(tlin_hbm.at[idx[s]], rows[s], gsem[s]).wait()

    def transpose(s):
        def body(k, carry):
            for u in range(8):
                r = k * 8 + u
                off = (r >> 7) * 1024 + (r & 127)
                offv = jnp.full((16,), off, jnp.int32)
                for cc in range(4):
                    vec = rows[s][r, pl.ds(cc * 16, 16)]
                    plsc.store_scatter(rT, [cpat[cc] + offv], vec)
            return carry
        lax.fori_loop(0, BW // 8, body, 0)

    def start_stores(l):
        for tc in range(8):
            pltpu.async_copy(
                rT.at[pl.ds(tc * TBW * 1024, TBW * 1024)],
                out_hbm.at[l, pl.ds(tc * (B * 8) + wid * (TBW * 1024), TBW * 1024)],
                ssem)

    def wait_stores(l):
        for tc in range(8):
            pltpu.make_async_copy(
                rT.at[pl.ds(tc * TBW * 1024, TBW * 1024)],
                out_hbm.at[l, pl.ds(tc * (B * 8) + wid * (TBW * 1024), TBW * 1024)],
                ssem).wait()

    start_idx(0, 0)
    wait_idx(0)
    start_idx(1, 1)
    start_gather(0)
    wait_idx(1)
    wait_gather(0)
    start_gather(1)
    start_idx(2, 0)
    transpose(0)
    start_stores(0)
    wait_gather(1)
    wait_idx(0)
    start_gather(0)
    start_idx(3, 1)
    wait_stores(0)
    transpose(1)
    start_stores(1)

    def body(g, carry):
        for u in range(2):
            l = g * 2 + u
            wait_gather(u)
            wait_idx(1 - u)
            start_gather(1 - u)
            start_idx(l + 2, u)
            wait_stores(l - 1)
            transpose(u)
            start_stores(l)
        return carry

    lax.fori_loop(1, L // 2, body, 0)

    wait_gather(0)
    wait_idx(1)
    wait_stores(L - 1)


def kernel(x, table):
    tlin = _relayout(table.T).reshape(VOCAB, HIDDEN)
    out5 = _gather(x.T.astype(jnp.int32), tlin)
    return (out5.reshape(L, 8, 128, 8, 128)
                .transpose(2, 4, 0, 1, 3)
                .reshape(B, L, HIDDEN))


# speedup vs baseline: 1.0075x; 1.0075x over previous
import functools

import jax
import jax.numpy as jnp
from jax import lax
from jax.experimental import pallas as pl
from jax.experimental.pallas import tpu as pltpu
from jax.experimental.pallas import tpu_sc as plsc

VOCAB = 1000000
HIDDEN = 64
B = 16384
L = 50

NUM_CORES = 2
NUM_SUBCORES = 16
NW = NUM_CORES * NUM_SUBCORES

CB = 400
NBLK = VOCAB // CB
VBLK = 80

BW = B // NW
TBW = BW // 128
PLANE = HIDDEN * B

_params = pltpu.CompilerParams(use_tc_tiling_on_sc=False, needs_layout_passes=False)
_mesh = plsc.VectorSubcoreMesh(core_axis_name="c", subcore_axis_name="s")


def _wid():
    return lax.axis_index("s") * NUM_CORES + lax.axis_index("c")


@functools.partial(
    pl.kernel,
    mesh=_mesh,
    out_type=jax.ShapeDtypeStruct((VOCAB * HIDDEN,), jnp.float32),
    scratch_types=[
        pltpu.VMEM((HIDDEN, CB), jnp.float32),
        pltpu.VMEM((HIDDEN, CB), jnp.float32),
        pltpu.VMEM((CB * HIDDEN,), jnp.float32),
        pltpu.VMEM((CB * HIDDEN,), jnp.float32),
        pltpu.SemaphoreType.DMA,
        pltpu.SemaphoreType.DMA,
        pltpu.SemaphoreType.DMA,
        pltpu.SemaphoreType.DMA,
    ],
    compiler_params=_params,
)
def _relayout(tT_hbm, tlin_hbm, b0, b1, t0, t1, r0, r1, w0, w1):
    wid = _wid()
    buf = (b0, b1)
    bufT = (t0, t1)
    rsem = (r0, r1)
    wsem = (w0, w1)
    ii = lax.iota(jnp.int32, 16)
    iota64 = ii * HIDDEN

    def c0(i):
        blk = jnp.minimum(wid + i * NW, NBLK - 1)
        return pl.multiple_of(blk * CB, 8)

    def start_read(i, s):
        for q in range(4):
            pltpu.async_copy(tT_hbm.at[pl.ds(q * 16, 16), pl.ds(c0(i), CB)],
                             buf[s].at[pl.ds(q * 16, 16), :], rsem[s])

    def wait_read(i, s):
        for q in range(4):
            pltpu.make_async_copy(tT_hbm.at[pl.ds(q * 16, 16), pl.ds(c0(i), CB)],
                                  buf[s].at[pl.ds(q * 16, 16), :], rsem[s]).wait()

    def transpose(s):
        def body(g, carry):
            for cu in range(4):
                c = g * 4 + cu
                basev = iota64 + jnp.full((16,), c, jnp.int32)
                for rr0 in range(0, CB // 16, 5):
                    wave = []
                    for rr in range(rr0, rr0 + 5):
                        vec = buf[s][c, pl.ds(rr * 16, 16)]
                        wave.append((basev + rr * 16 * HIDDEN, vec))
                    for idxv, vec in wave:
                        plsc.store_scatter(bufT[s], [idxv], vec)
            return carry
        lax.fori_loop(0, HIDDEN // 4, body, 0)

    def start_write(i, s):
        pltpu.async_copy(bufT[s], tlin_hbm.at[pl.ds(c0(i) * HIDDEN, CB * HIDDEN)], wsem[s])

    def wait_write(i, s):
        pltpu.make_async_copy(bufT[s], tlin_hbm.at[pl.ds(c0(i) * HIDDEN, CB * HIDDEN)],
                              wsem[s]).wait()

    start_read(0, 0)
    wait_read(0, 0)
    start_read(1, 1)
    transpose(0)
    start_write(0, 0)
    wait_read(1, 1)
    start_read(2, 0)
    transpose(1)
    start_write(1, 1)

    def body(g, carry):
        for u in range(2):
            i = g * 2 + u
            wait_read(i, u)
            start_read(i + 1, 1 - u)
            wait_write(i - 2, u)
            transpose(u)
            start_write(i, u)
        return carry

    lax.fori_loop(1, VBLK // 2, body, 0)

    wait_write(VBLK - 2, 0)
    wait_write(VBLK - 1, 1)
    wait_read(VBLK, 0)


@functools.partial(
    pl.kernel,
    mesh=_mesh,
    out_type=jax.ShapeDtypeStruct((L, PLANE), jnp.float32),
    scratch_types=[
        pltpu.VMEM((BW, L), jnp.int32),
        pltpu.VMEM((BW,), jnp.int32),
        pltpu.VMEM((BW,), jnp.int32),
        pltpu.VMEM((BW, HIDDEN), jnp.float32),
        pltpu.VMEM((BW, HIDDEN), jnp.float32),
        pltpu.VMEM((BW * HIDDEN,), jnp.float32),
        pltpu.SemaphoreType.DMA,
        pltpu.SemaphoreType.DMA,
        pltpu.SemaphoreType.DMA,
    ],
    compiler_params=_params,
)
def _gather(x_hbm, tlin_hbm, out_hbm,
            xb, i0, i1, rowsA, rowsB, rT, gs0, gs1, ssem):
    wid = _wid()
    base = pl.multiple_of(wid * BW, 8)
    idx = (i0, i1)
    rows = (rowsA, rowsB)
    gsem = (gs0, gs1)
    ii = lax.iota(jnp.int32, 16)
    cpat = []
    for cc in range(4):
        cv = ii + cc * 16
        cpat.append((cv // 8) * (TBW * 1024) + (cv % 8) * 128)

    def extract(l, s):
        lc = jnp.minimum(l, L - 1)
        colv = jnp.full((16,), lc, jnp.int32)
        for k in range(0, BW // 16, 8):
            wave = []
            for u in range(8):
                rowv = ii + (k + u) * 16
                wave.append(plsc.load_gather(xb, [rowv, colv]))
            for u in range(8):
                idx[s][pl.ds((k + u) * 16, 16)] = wave[u]

    def start_gather(s):
        pltpu.async_copy(tlin_hbm.at[idx[s]], rows[s], gsem[s])

    def wait_gather(s):
        pltpu.make_async_copy(tlin_hbm.at[idx[s]], rows[s], gsem[s]).wait()

    def transpose(s):
        def body(k, carry):
            r0 = k * 8
            for u2 in range(0, 8, 2):
                wave = []
                for u in (u2, u2 + 1):
                    r = r0 + u
                    off = (r >> 7) * 1024 + (r & 127)
                    offv = jnp.full((16,), off, jnp.int32)
                    for cc in range(4):
                        vec = rows[s][r, pl.ds(cc * 16, 16)]
                        wave.append((cpat[cc] + offv, vec))
                for idxv, vec in wave:
                    plsc.store_scatter(rT, [idxv], vec)
            return carry
        lax.fori_loop(0, BW // 8, body, 0)

    def start_stores(l):
        for tc in range(8):
            pltpu.async_copy(
                rT.at[pl.ds(tc * TBW * 1024, TBW * 1024)],
                out_hbm.at[l, pl.ds(tc * (B * 8) + wid * (TBW * 1024), TBW * 1024)],
                ssem)

    def wait_stores(l):
        for tc in range(8):
            pltpu.make_async_copy(
                rT.at[pl.ds(tc * TBW * 1024, TBW * 1024)],
                out_hbm.at[l, pl.ds(tc * (B * 8) + wid * (TBW * 1024), TBW * 1024)],
                ssem).wait()

    pltpu.sync_copy(x_hbm.at[pl.ds(base, BW), :], xb)

    extract(0, 0)
    start_gather(0)
    extract(1, 1)
    wait_gather(0)
    start_gather(1)
    transpose(0)
    start_stores(0)
    wait_gather(1)
    extract(2, 0)
    start_gather(0)
    wait_stores(0)
    transpose(1)
    start_stores(1)

    def body(g, carry):
        for u in range(2):
            l = g * 2 + u
            wait_gather(u)
            extract(l + 1, 1 - u)
            start_gather(1 - u)
            wait_stores(l - 1)
            transpose(u)
            start_stores(l)
        return carry

    lax.fori_loop(1, L // 2, body, 0)

    wait_gather(0)
    wait_stores(L - 1)


def kernel(x, table):
    tlin = _relayout(table.T).reshape(VOCAB, HIDDEN)
    out5 = _gather(x.astype(jnp.int32), tlin)
    return (out5.reshape(L, 8, 128, 8, 128)
                .transpose(2, 4, 0, 1, 3)
                .reshape(B, L, HIDDEN))
